# Initial kernel scaffold; baseline (speedup 1.0000x reference)
import jax
import jax.numpy as jnp
from jax.experimental import pallas as pl


def kernel(data, edge_index, edge_weight, W1, att_src1, att_dst1, b1, W2, att_src2, att_dst2, b2):
    raise NotImplementedError("write your pallas kernel here")


# speedup vs baseline: 30.2408x; 30.2408x over previous
import functools

import jax
import jax.numpy as jnp
from jax import lax
from jax.experimental import pallas as pl
from jax.experimental.pallas import tpu as pltpu
from jax.experimental.pallas import tpu_sc as plsc

N = 10000
NP = 10240
E = 320000
EP = 344064
S12 = EP // 16
S3 = EP // 32
CH = 128
NEG = 0.2
PAD_LA = -1e30


def _tc1_body(x_ref, w_ref, avs_ref, avd_ref, ew_ref,
              h_ref, as_ref, ad_ref, la_ref):
    h = jnp.dot(x_ref[...], w_ref[...], preferred_element_type=jnp.float32)
    h_ref[...] = h
    as_ref[...] = jnp.sum(h * avs_ref[...][None, :], axis=1)
    ad_ref[...] = jnp.sum(h * avd_ref[...][None, :], axis=1)
    la_ref[...] = jnp.log2(ew_ref[...])


def _tc2_body(p_ref, b_ref, w_ref, avs_ref, avd_ref,
              h_ref, as_ref, ad_ref):
    o = p_ref[...] + b_ref[...][None, :]
    o = jnp.maximum(o, 0.0)
    h = jnp.dot(o, w_ref[...], preferred_element_type=jnp.float32)
    h_ref[...] = h
    as_ref[...] = jnp.sum(h * avs_ref[...][None, :], axis=1)
    ad_ref[...] = jnp.sum(h * avd_ref[...][None, :], axis=1)


def _tc3_body(p_ref, b_ref, o_ref):
    o_ref[...] = p_ref[0] + p_ref[1] + b_ref[...][None, :]


def _edge_body(HS, split_cols, src_hbm, dst_hbm, la_hbm, as_hbm, ad_hbm,
               ha_hbm, hb_hbm, rid_hbm, out_hbm,
               as_tab, ad_tab, dn_tab, src_buf, dst_buf, la_buf,
               rows, zrows, zbuf, tmp16, mxl, coef, rid_v, sem,
               dn_sh, mx_sh, out_sh):
    c = lax.axis_index("c")
    t = lax.axis_index("s")
    JH = HS // 16
    zero16 = jnp.zeros((16,), jnp.float32)

    pltpu.sync_copy(as_hbm, as_tab)
    pltpu.sync_copy(ad_hbm, ad_tab)
    pltpu.sync_copy(rid_hbm, rid_v)
    nrow = S12 // 128
    pltpu.sync_copy(src_hbm.at[pl.ds(t * nrow, nrow)], src_buf)
    pltpu.sync_copy(dst_hbm.at[pl.ds(t * nrow, nrow)], dst_buf)
    pltpu.sync_copy(la_hbm.at[pl.ds(t * S12, S12)], la_buf)

    def _zb(i, carry):
        zbuf[i, pl.ds(0, 16)] = zero16
        return carry
    lax.fori_loop(0, 40, _zb, 0)

    def _zr(i, carry):
        e = i // JH
        j = i % JH
        zrows[e, pl.ds(j * 16, 16)] = zero16
        return carry
    lax.fori_loop(0, 128 * JH, _zr, 0)

    def _zd(i, carry):
        dn_tab[i, pl.ds(0, 16)] = zero16
        return carry
    lax.fori_loop(0, NP // 16, _zd, 0)

    pltpu.sync_copy(zbuf, dn_sh.at[pl.ds(t * 40, 40)])
    for k in range(5):
        pltpu.sync_copy(zrows.at[pl.ds(0, 128)],
                        out_sh.at[pl.ds(t * 640 + k * 128, 128)])

    def _p1(r, vmax):
        for c8 in range(8):
            sv = src_buf[r, pl.ds(c8 * 16, 16)]
            dv = dst_buf[r, pl.ds(c8 * 16, 16)]
            x = plsc.load_gather(as_tab, [sv]) + plsc.load_gather(ad_tab, [dv])
            x = jnp.where(x >= 0.0, x, NEG * x)
            a = x + la_buf[pl.ds(r * 128 + c8 * 16, 16)]
            la_buf[pl.ds(r * 128 + c8 * 16, 16)] = a
            vmax = jnp.maximum(vmax, a)
        return vmax
    vmax = lax.fori_loop(0, nrow, _p1,
                         jnp.full((16,), -3e38, jnp.float32))
    tmp16[...] = vmax
    pltpu.sync_copy(tmp16, mx_sh.at[t])

    plsc.subcore_barrier()

    pltpu.sync_copy(mx_sh, mxl)
    def _mx(i, m):
        return jnp.maximum(m, mxl[i])
    gmax = jnp.max(lax.fori_loop(0, 16, _mx,
                                 jnp.full((16,), -3e38, jnp.float32)))
    gv = jnp.full((16,), gmax, jnp.float32)

    def _p2(r, carry):
        for c8 in range(8):
            a = la_buf[pl.ds(r * 128 + c8 * 16, 16)]
            dv = dst_buf[r, pl.ds(c8 * 16, 16)]
            ex = jnp.exp(a - gv)
            la_buf[pl.ds(r * 128 + c8 * 16, 16)] = ex
            plsc.addupdate_scatter(dn_tab, [dv >> 4, dv & 15], ex)
        return carry
    lax.fori_loop(0, nrow, _p2, 0)

    for k in range(5):
        pltpu.sync_copy(dn_tab.at[pl.ds(k * 128, 128)],
                        dn_sh.at[rid_v.at[k]], add=True)

    plsc.subcore_barrier()

    pltpu.sync_copy(dn_sh, dn_tab)

    off0 = 0 if split_cols else c * (S3 // CH)
    nch = (S12 // CH) if split_cols else (S3 // CH)

    def _p3(k, carry):
        r3 = off0 + k

        @pl.when(c == 0)
        def _g0():
            pltpu.async_copy(ha_hbm.at[src_buf.at[r3]], rows, sem).wait()

        @pl.when(c == 1)
        def _g1():
            pltpu.async_copy(hb_hbm.at[src_buf.at[r3]], rows, sem).wait()

        def _cf(v, carry2):
            ex = la_buf[pl.ds(r3 * 128 + v * 16, 16)]
            dv = dst_buf[r3, pl.ds(v * 16, 16)]
            dn = plsc.load_gather(dn_tab, [dv >> 4, dv & 15])
            coef[pl.ds(v * 16, 16)] = ex / (dn + 1e-16)
            return carry2
        lax.fori_loop(0, CH // 16, _cf, 0)

        def _sc(v, carry2):
            cv16 = coef[pl.ds(v * 16, 16)]
            for l in range(16):
                e = v * 16 + l
                cv = jnp.full((16,), cv16[l], jnp.float32)
                for j in range(JH):
                    rows[e, pl.ds(j * 16, 16)] = (
                        rows[e, pl.ds(j * 16, 16)] * cv)
            return carry2
        lax.fori_loop(0, CH // 16, _sc, 0)

        pltpu.sync_copy(rows, out_sh.at[dst_buf.at[r3]], add=True)
        return carry
    lax.fori_loop(0, nch, _p3, 0)

    plsc.subcore_barrier()

    pltpu.sync_copy(out_sh.at[pl.ds(t * 640, 640)],
                    out_hbm.at[c, pl.ds(t * 640, 640)])


def _make_edge_kernel(HS, split_cols):
    mesh = plsc.VectorSubcoreMesh(core_axis_name="c", subcore_axis_name="s")
    return pl.kernel(
        functools.partial(_edge_body, HS, split_cols),
        out_type=jax.ShapeDtypeStruct((2, NP, HS), jnp.float32),
        mesh=mesh,
        compiler_params=pltpu.CompilerParams(needs_layout_passes=False, use_tc_tiling_on_sc=False),
        scratch_types=[
            pltpu.VMEM((NP,), jnp.float32),
            pltpu.VMEM((NP,), jnp.float32),
            pltpu.VMEM((NP // 16, 16), jnp.float32),
            pltpu.VMEM((S12 // 128, 128), jnp.int32),
            pltpu.VMEM((S12 // 128, 128), jnp.int32),
            pltpu.VMEM((S12,), jnp.float32),
            pltpu.VMEM((CH, HS), jnp.float32),
            pltpu.VMEM((128, HS), jnp.float32),
            pltpu.VMEM((40, 16), jnp.float32),
            pltpu.VMEM((16,), jnp.float32),
            pltpu.VMEM((16, 16), jnp.float32),
            pltpu.VMEM((CH,), jnp.float32),
            pltpu.VMEM((5, 128), jnp.int32),
            pltpu.SemaphoreType.DMA,
            pltpu.VMEM_SHARED((NP // 16, 16), jnp.float32),
            pltpu.VMEM_SHARED((16, 16), jnp.float32),
            pltpu.VMEM_SHARED((NP, HS), jnp.float32),
        ],
    )


_edge_kernel_l1 = _make_edge_kernel(32, True)
_edge_kernel_l2 = _make_edge_kernel(16, False)

_tc1 = pl.pallas_call(
    _tc1_body,
    out_shape=[
        jax.ShapeDtypeStruct((NP, 64), jnp.float32),
        jax.ShapeDtypeStruct((NP,), jnp.float32),
        jax.ShapeDtypeStruct((NP,), jnp.float32),
        jax.ShapeDtypeStruct((E,), jnp.float32),
    ],
)

_tc2 = pl.pallas_call(
    _tc2_body,
    out_shape=[
        jax.ShapeDtypeStruct((NP, 16), jnp.float32),
        jax.ShapeDtypeStruct((NP,), jnp.float32),
        jax.ShapeDtypeStruct((NP,), jnp.float32),
    ],
)

_tc3 = pl.pallas_call(
    _tc3_body,
    out_shape=jax.ShapeDtypeStruct((NP, 16), jnp.float32),
)


def kernel(data, edge_index, edge_weight, W1, att_src1, att_dst1, b1,
           W2, att_src2, att_dst2, b2):
    xp = jnp.pad(data, ((0, NP - N), (0, 0)))

    h1, as1, ad1, la = _tc1(xp, W1, att_src1, att_dst1, edge_weight)

    loop = jnp.arange(N, dtype=jnp.int32)
    padi = jnp.zeros((EP - E - N,), dtype=jnp.int32)
    src = jnp.concatenate([edge_index[0], loop, padi])
    dst = jnp.concatenate([edge_index[1], loop, padi])
    la_full = jnp.concatenate([
        la, jnp.zeros((N,), jnp.float32),
        jnp.full((EP - E - N,), PAD_LA, jnp.float32),
    ])

    rid = jnp.arange(NP // 16, dtype=jnp.int32).reshape(5, 128)
    src2 = src.reshape(EP // 128, 128)
    dst2 = dst.reshape(EP // 128, 128)
    h1n = h1[:N]
    p1 = _edge_kernel_l1(src2, dst2, la_full, as1, ad1,
                         h1n[:, :32], h1n[:, 32:], rid)
    agg1 = jnp.concatenate([p1[0], p1[1]], axis=1)
    h2, as2, ad2 = _tc2(agg1, b1, W2, att_src2, att_dst2)
    h2n = h2[:N]
    p2 = _edge_kernel_l2(src2, dst2, la_full, as2, ad2, h2n, h2n, rid)
    out = _tc3(p2, b2)
    return out[:N]


# speedup vs baseline: 38.3369x; 1.2677x over previous
import functools

import jax
import jax.numpy as jnp
from jax import lax
from jax.experimental import pallas as pl
from jax.experimental.pallas import tpu as pltpu
from jax.experimental.pallas import tpu_sc as plsc

N = 10000
NP = 10240
E = 320000
EP = 344064
S12 = EP // 16
S3 = EP // 32
CH = 128
NEG = 0.2
PAD_LA = -1e30


def _tc1_body(x_ref, w_ref, avs_ref, avd_ref, ew_ref,
              h_ref, as_ref, ad_ref, la_ref):
    h = jnp.dot(x_ref[...], w_ref[...], preferred_element_type=jnp.float32)
    h_ref[...] = h
    as_ref[...] = jnp.sum(h * avs_ref[...][None, :], axis=1)
    ad_ref[...] = jnp.sum(h * avd_ref[...][None, :], axis=1)
    la_ref[...] = jnp.log2(ew_ref[...])


def _tc2_body(p_ref, b_ref, w_ref, avs_ref, avd_ref,
              h_ref, as_ref, ad_ref):
    o = p_ref[...] + b_ref[...][None, :]
    o = jnp.maximum(o, 0.0)
    h = jnp.dot(o, w_ref[...], preferred_element_type=jnp.float32)
    h_ref[...] = h
    as_ref[...] = jnp.sum(h * avs_ref[...][None, :], axis=1)
    ad_ref[...] = jnp.sum(h * avd_ref[...][None, :], axis=1)


def _tc3_body(p_ref, b_ref, o_ref):
    o_ref[...] = p_ref[0] + p_ref[1] + b_ref[...][None, :]


def _edge_body(HS, split_cols, src_hbm, dst_hbm, la_hbm, as_hbm, ad_hbm,
               ha_hbm, hb_hbm, rid_hbm, out_hbm,
               as_tab, ad_tab, dn_tab, src_buf, dst_buf, la_buf,
               rows, rows2, zrows, zbuf, tmp16, mxl, rid_v, sem, sem2,
               dn_sh, mx_sh, out_sh):
    c = lax.axis_index("c")
    t = lax.axis_index("s")
    JH = HS // 16
    zero16 = jnp.zeros((16,), jnp.float32)

    pltpu.sync_copy(as_hbm, as_tab)
    pltpu.sync_copy(ad_hbm, ad_tab)
    pltpu.sync_copy(rid_hbm, rid_v)
    nrow = S12 // 128
    pltpu.sync_copy(src_hbm.at[pl.ds(t * nrow, nrow)], src_buf)
    pltpu.sync_copy(dst_hbm.at[pl.ds(t * nrow, nrow)], dst_buf)
    pltpu.sync_copy(la_hbm.at[pl.ds(t * S12, S12)], la_buf)

    def _zb(i, carry):
        zbuf[i, pl.ds(0, 16)] = zero16
        return carry
    lax.fori_loop(0, 40, _zb, 0)

    def _zr(i, carry):
        e = i // JH
        j = i % JH
        zrows[e, pl.ds(j * 16, 16)] = zero16
        return carry
    lax.fori_loop(0, 128 * JH, _zr, 0)

    def _zd(i, carry):
        dn_tab[i, pl.ds(0, 16)] = zero16
        return carry
    lax.fori_loop(0, NP // 16, _zd, 0)

    pltpu.sync_copy(zbuf, dn_sh.at[pl.ds(t * 40, 40)])
    for k in range(5):
        pltpu.sync_copy(zrows.at[pl.ds(0, 128)],
                        out_sh.at[pl.ds(t * 640 + k * 128, 128)])

    def _p1(r, vmax):
        for c8 in range(8):
            sv = src_buf[r, pl.ds(c8 * 16, 16)]
            dv = dst_buf[r, pl.ds(c8 * 16, 16)]
            x = plsc.load_gather(as_tab, [sv]) + plsc.load_gather(ad_tab, [dv])
            x = jnp.where(x >= 0.0, x, NEG * x)
            a = x + la_buf[pl.ds(r * 128 + c8 * 16, 16)]
            la_buf[pl.ds(r * 128 + c8 * 16, 16)] = a
            vmax = jnp.maximum(vmax, a)
        return vmax
    vmax = lax.fori_loop(0, nrow, _p1,
                         jnp.full((16,), -3e38, jnp.float32))
    tmp16[...] = vmax
    pltpu.sync_copy(tmp16, mx_sh.at[t])

    plsc.subcore_barrier()

    pltpu.sync_copy(mx_sh, mxl)
    def _mx(i, m):
        return jnp.maximum(m, mxl[i])
    gmax = jnp.max(lax.fori_loop(0, 16, _mx,
                                 jnp.full((16,), -3e38, jnp.float32)))
    gv = jnp.full((16,), gmax, jnp.float32)

    def _p2(r, carry):
        for c8 in range(8):
            a = la_buf[pl.ds(r * 128 + c8 * 16, 16)]
            dv = dst_buf[r, pl.ds(c8 * 16, 16)]
            ex = jnp.exp(a - gv)
            la_buf[pl.ds(r * 128 + c8 * 16, 16)] = ex
            plsc.addupdate_scatter(dn_tab, [dv >> 4, dv & 15], ex)
        return carry
    lax.fori_loop(0, nrow, _p2, 0)

    for k in range(5):
        pltpu.sync_copy(dn_tab.at[pl.ds(k * 128, 128)],
                        dn_sh.at[rid_v.at[k]], add=True)

    plsc.subcore_barrier()

    pltpu.sync_copy(dn_sh, dn_tab)

    def _pc(r, carry):
        for c8 in range(8):
            ex = la_buf[pl.ds(r * 128 + c8 * 16, 16)]
            dv = dst_buf[r, pl.ds(c8 * 16, 16)]
            dn = plsc.load_gather(dn_tab, [dv >> 4, dv & 15])
            la_buf[pl.ds(r * 128 + c8 * 16, 16)] = ex / (dn + 1e-16)
        return carry
    lax.fori_loop(0, nrow, _pc, 0)

    off0 = 0 if split_cols else c * (S3 // CH)
    nch = (S12 // CH) if split_cols else (S3 // CH)
    h_table = [ha_hbm, hb_hbm]
    rbufs = [rows, rows2]
    sems = [sem, sem2]

    def _issue(r3, b):
        @pl.when(c == 0)
        def _g0():
            pltpu.async_copy(h_table[0].at[src_buf.at[r3]], rbufs[b], sems[b])

        @pl.when(c == 1)
        def _g1():
            pltpu.async_copy(h_table[1].at[src_buf.at[r3]], rbufs[b], sems[b])

    def _process(r3, b):
        pltpu.make_async_copy(h_table[0].at[src_buf.at[r3]],
                              rbufs[b], sems[b]).wait()

        def _sc(v, carry2):
            cv16 = la_buf[pl.ds(r3 * 128 + v * 16, 16)]
            for l in range(16):
                e = v * 16 + l
                cv = jnp.full((16,), cv16[l], jnp.float32)
                for j in range(JH):
                    rbufs[b][e, pl.ds(j * 16, 16)] = (
                        rbufs[b][e, pl.ds(j * 16, 16)] * cv)
            return carry2
        lax.fori_loop(0, CH // 16, _sc, 0)

        pltpu.sync_copy(rbufs[b], out_sh.at[dst_buf.at[r3]], add=True)

    _issue(off0, 0)

    def _p3(k2, carry):
        k0 = k2 * 2
        _issue(off0 + k0 + 1, 1)
        _process(off0 + k0, 0)

        @pl.when(k0 + 2 < nch)
        def _nx():
            _issue(off0 + k0 + 2, 0)
        _process(off0 + k0 + 1, 1)
        return carry
    lax.fori_loop(0, nch // 2, _p3, 0)

    plsc.subcore_barrier()

    pltpu.sync_copy(out_sh.at[pl.ds(t * 640, 640)],
                    out_hbm.at[c, pl.ds(t * 640, 640)])


def _make_edge_kernel(HS, split_cols):
    mesh = plsc.VectorSubcoreMesh(core_axis_name="c", subcore_axis_name="s")
    return pl.kernel(
        functools.partial(_edge_body, HS, split_cols),
        out_type=jax.ShapeDtypeStruct((2, NP, HS), jnp.float32),
        mesh=mesh,
        compiler_params=pltpu.CompilerParams(needs_layout_passes=False, use_tc_tiling_on_sc=False),
        scratch_types=[
            pltpu.VMEM((NP,), jnp.float32),
            pltpu.VMEM((NP,), jnp.float32),
            pltpu.VMEM((NP // 16, 16), jnp.float32),
            pltpu.VMEM((S12 // 128, 128), jnp.int32),
            pltpu.VMEM((S12 // 128, 128), jnp.int32),
            pltpu.VMEM((S12,), jnp.float32),
            pltpu.VMEM((CH, HS), jnp.float32),
            pltpu.VMEM((CH, HS), jnp.float32),
            pltpu.VMEM((128, HS), jnp.float32),
            pltpu.VMEM((40, 16), jnp.float32),
            pltpu.VMEM((16,), jnp.float32),
            pltpu.VMEM((16, 16), jnp.float32),
            pltpu.VMEM((5, 128), jnp.int32),
            pltpu.SemaphoreType.DMA,
            pltpu.SemaphoreType.DMA,
            pltpu.VMEM_SHARED((NP // 16, 16), jnp.float32),
            pltpu.VMEM_SHARED((16, 16), jnp.float32),
            pltpu.VMEM_SHARED((NP, HS), jnp.float32),
        ],
    )


_edge_kernel_l1 = _make_edge_kernel(32, True)
_edge_kernel_l2 = _make_edge_kernel(16, False)

_tc1 = pl.pallas_call(
    _tc1_body,
    out_shape=[
        jax.ShapeDtypeStruct((NP, 64), jnp.float32),
        jax.ShapeDtypeStruct((NP,), jnp.float32),
        jax.ShapeDtypeStruct((NP,), jnp.float32),
        jax.ShapeDtypeStruct((E,), jnp.float32),
    ],
)

_tc2 = pl.pallas_call(
    _tc2_body,
    out_shape=[
        jax.ShapeDtypeStruct((NP, 16), jnp.float32),
        jax.ShapeDtypeStruct((NP,), jnp.float32),
        jax.ShapeDtypeStruct((NP,), jnp.float32),
    ],
)

_tc3 = pl.pallas_call(
    _tc3_body,
    out_shape=jax.ShapeDtypeStruct((NP, 16), jnp.float32),
)


def kernel(data, edge_index, edge_weight, W1, att_src1, att_dst1, b1,
           W2, att_src2, att_dst2, b2):
    xp = jnp.pad(data, ((0, NP - N), (0, 0)))

    h1, as1, ad1, la = _tc1(xp, W1, att_src1, att_dst1, edge_weight)

    loop = jnp.arange(N, dtype=jnp.int32)
    padi = jnp.zeros((EP - E - N,), dtype=jnp.int32)
    src = jnp.concatenate([edge_index[0], loop, padi])
    dst = jnp.concatenate([edge_index[1], loop, padi])
    la_full = jnp.concatenate([
        la, jnp.zeros((N,), jnp.float32),
        jnp.full((EP - E - N,), PAD_LA, jnp.float32),
    ])

    rid = jnp.arange(NP // 16, dtype=jnp.int32).reshape(5, 128)
    src2 = src.reshape(EP // 128, 128)
    dst2 = dst.reshape(EP // 128, 128)
    h1n = h1[:N]
    p1 = _edge_kernel_l1(src2, dst2, la_full, as1, ad1,
                         h1n[:, :32], h1n[:, 32:], rid)
    agg1 = jnp.concatenate([p1[0], p1[1]], axis=1)
    h2, as2, ad2 = _tc2(agg1, b1, W2, att_src2, att_dst2)
    h2n = h2[:N]
    p2 = _edge_kernel_l2(src2, dst2, la_full, as2, ad2, h2n, h2n, rid)
    out = _tc3(p2, b2)
    return out[:N]


# speedup vs baseline: 38.3423x; 1.0001x over previous
import functools

import jax
import jax.numpy as jnp
from jax import lax
from jax.experimental import pallas as pl
from jax.experimental.pallas import tpu as pltpu
from jax.experimental.pallas import tpu_sc as plsc

N = 10000
NP = 10240
E = 320000
EP = 344064
S12 = EP // 16
S3 = EP // 32
CH = 128
NEG = 0.2
PAD_LA = -1e30


def _tc1_body(x_ref, w_ref, avs_ref, avd_ref, ew_ref,
              h_ref, as_ref, ad_ref, la_ref):
    h = jnp.dot(x_ref[...], w_ref[...], preferred_element_type=jnp.float32)
    h_ref[...] = h
    as_ref[...] = jnp.sum(h * avs_ref[...][None, :], axis=1)
    ad_ref[...] = jnp.sum(h * avd_ref[...][None, :], axis=1)
    la_ref[...] = jnp.log2(ew_ref[...])


def _tc2_body(p_ref, b_ref, w_ref, avs_ref, avd_ref,
              h_ref, as_ref, ad_ref):
    o = p_ref[...] + b_ref[...][None, :]
    o = jnp.maximum(o, 0.0)
    h = jnp.dot(o, w_ref[...], preferred_element_type=jnp.float32)
    h_ref[...] = h
    as_ref[...] = jnp.sum(h * avs_ref[...][None, :], axis=1)
    ad_ref[...] = jnp.sum(h * avd_ref[...][None, :], axis=1)


def _tc3_body(p_ref, b_ref, o_ref):
    o_ref[...] = p_ref[0] + p_ref[1] + b_ref[...][None, :]


def _edge_body(HS, split_cols, src_hbm, dst_hbm, la_hbm, as_hbm, ad_hbm,
               ha_hbm, hb_hbm, rid_hbm, out_hbm,
               as_tab, ad_tab, dn_tab, src_buf, dst_buf, la_buf,
               rows, rows2, zrows, zbuf, tmp16, mxl, rid_v, sem, sem2,
               dn_sh, mx_sh, out_sh):
    c = lax.axis_index("c")
    t = lax.axis_index("s")
    JH = HS // 16
    zero16 = jnp.zeros((16,), jnp.float32)

    _scope_stage = jax.named_scope("p0_stage")
    _scope_stage.__enter__()
    pltpu.sync_copy(as_hbm, as_tab)
    pltpu.sync_copy(ad_hbm, ad_tab)
    pltpu.sync_copy(rid_hbm, rid_v)
    nrow = S12 // 128
    pltpu.sync_copy(src_hbm.at[pl.ds(t * nrow, nrow)], src_buf)
    pltpu.sync_copy(dst_hbm.at[pl.ds(t * nrow, nrow)], dst_buf)
    pltpu.sync_copy(la_hbm.at[pl.ds(t * S12, S12)], la_buf)

    def _zb(i, carry):
        zbuf[i, pl.ds(0, 16)] = zero16
        return carry
    lax.fori_loop(0, 40, _zb, 0)

    def _zr(i, carry):
        e = i // JH
        j = i % JH
        zrows[e, pl.ds(j * 16, 16)] = zero16
        return carry
    lax.fori_loop(0, 128 * JH, _zr, 0)

    def _zd(i, carry):
        dn_tab[i, pl.ds(0, 16)] = zero16
        return carry
    lax.fori_loop(0, NP // 16, _zd, 0)

    pltpu.sync_copy(zbuf, dn_sh.at[pl.ds(t * 40, 40)])
    for k in range(5):
        pltpu.sync_copy(zrows.at[pl.ds(0, 128)],
                        out_sh.at[pl.ds(t * 640 + k * 128, 128)])

    _scope_stage.__exit__(None, None, None)

    _s1 = jax.named_scope("p1_alpha")
    _s1.__enter__()
    def _p1(r, vmax):
        for c8 in range(8):
            sv = src_buf[r, pl.ds(c8 * 16, 16)]
            dv = dst_buf[r, pl.ds(c8 * 16, 16)]
            x = plsc.load_gather(as_tab, [sv]) + plsc.load_gather(ad_tab, [dv])
            x = jnp.where(x >= 0.0, x, NEG * x)
            a = x + la_buf[pl.ds(r * 128 + c8 * 16, 16)]
            la_buf[pl.ds(r * 128 + c8 * 16, 16)] = a
            vmax = jnp.maximum(vmax, a)
        return vmax
    vmax = lax.fori_loop(0, nrow, _p1,
                         jnp.full((16,), -3e38, jnp.float32))
    tmp16[...] = vmax
    pltpu.sync_copy(tmp16, mx_sh.at[t])

    _s1.__exit__(None, None, None)
    _sb1 = jax.named_scope("b1_barrier")
    _sb1.__enter__()
    plsc.subcore_barrier()
    _sb1.__exit__(None, None, None)

    pltpu.sync_copy(mx_sh, mxl)
    def _mx(i, m):
        return jnp.maximum(m, mxl[i])
    gmax = jnp.max(lax.fori_loop(0, 16, _mx,
                                 jnp.full((16,), -3e38, jnp.float32)))
    gv = jnp.full((16,), gmax, jnp.float32)

    _s2 = jax.named_scope("p2_exp")
    _s2.__enter__()
    def _p2(r, carry):
        for c8 in range(8):
            a = la_buf[pl.ds(r * 128 + c8 * 16, 16)]
            dv = dst_buf[r, pl.ds(c8 * 16, 16)]
            ex = jnp.exp(a - gv)
            la_buf[pl.ds(r * 128 + c8 * 16, 16)] = ex
            plsc.addupdate_scatter(dn_tab, [dv >> 4, dv & 15], ex)
        return carry
    lax.fori_loop(0, nrow, _p2, 0)

    for k in range(5):
        pltpu.sync_copy(dn_tab.at[pl.ds(k * 128, 128)],
                        dn_sh.at[rid_v.at[k]], add=True)

    _s2.__exit__(None, None, None)
    _sb2 = jax.named_scope("b2_barrier")
    _sb2.__enter__()
    plsc.subcore_barrier()
    _sb2.__exit__(None, None, None)

    pltpu.sync_copy(dn_sh, dn_tab)

    _s25 = jax.named_scope("p25_coef")
    _s25.__enter__()
    def _pc(r, carry):
        for c8 in range(8):
            ex = la_buf[pl.ds(r * 128 + c8 * 16, 16)]
            dv = dst_buf[r, pl.ds(c8 * 16, 16)]
            dn = plsc.load_gather(dn_tab, [dv >> 4, dv & 15])
            la_buf[pl.ds(r * 128 + c8 * 16, 16)] = ex / (dn + 1e-16)
        return carry
    lax.fori_loop(0, nrow, _pc, 0)

    _s25.__exit__(None, None, None)
    _s3 = jax.named_scope("p3_rows")
    _s3.__enter__()
    off0 = 0 if split_cols else c * (S3 // CH)
    nch = (S12 // CH) if split_cols else (S3 // CH)
    h_table = [ha_hbm, hb_hbm]
    rbufs = [rows, rows2]
    sems = [sem, sem2]

    def _issue(r3, b):
        @pl.when(c == 0)
        def _g0():
            pltpu.async_copy(h_table[0].at[src_buf.at[r3]], rbufs[b], sems[b])

        @pl.when(c == 1)
        def _g1():
            pltpu.async_copy(h_table[1].at[src_buf.at[r3]], rbufs[b], sems[b])

    def _process(r3, b):
        pltpu.make_async_copy(h_table[0].at[src_buf.at[r3]],
                              rbufs[b], sems[b]).wait()

        def _sc(v, carry2):
            cv16 = la_buf[pl.ds(r3 * 128 + v * 16, 16)]
            for l in range(16):
                e = v * 16 + l
                cv = jnp.full((16,), cv16[l], jnp.float32)
                for j in range(JH):
                    rbufs[b][e, pl.ds(j * 16, 16)] = (
                        rbufs[b][e, pl.ds(j * 16, 16)] * cv)
            return carry2
        lax.fori_loop(0, CH // 16, _sc, 0)

        pltpu.sync_copy(rbufs[b], out_sh.at[dst_buf.at[r3]], add=True)

    _issue(off0, 0)

    def _p3(k2, carry):
        k0 = k2 * 2
        _issue(off0 + k0 + 1, 1)
        _process(off0 + k0, 0)

        @pl.when(k0 + 2 < nch)
        def _nx():
            _issue(off0 + k0 + 2, 0)
        _process(off0 + k0 + 1, 1)
        return carry
    lax.fori_loop(0, nch // 2, _p3, 0)

    _s3.__exit__(None, None, None)
    plsc.subcore_barrier()

    pltpu.sync_copy(out_sh.at[pl.ds(t * 640, 640)],
                    out_hbm.at[c, pl.ds(t * 640, 640)])


def _make_edge_kernel(HS, split_cols):
    mesh = plsc.VectorSubcoreMesh(core_axis_name="c", subcore_axis_name="s")
    return pl.kernel(
        functools.partial(_edge_body, HS, split_cols),
        out_type=jax.ShapeDtypeStruct((2, NP, HS), jnp.float32),
        mesh=mesh,
        compiler_params=pltpu.CompilerParams(needs_layout_passes=False, use_tc_tiling_on_sc=False),
        scratch_types=[
            pltpu.VMEM((NP,), jnp.float32),
            pltpu.VMEM((NP,), jnp.float32),
            pltpu.VMEM((NP // 16, 16), jnp.float32),
            pltpu.VMEM((S12 // 128, 128), jnp.int32),
            pltpu.VMEM((S12 // 128, 128), jnp.int32),
            pltpu.VMEM((S12,), jnp.float32),
            pltpu.VMEM((CH, HS), jnp.float32),
            pltpu.VMEM((CH, HS), jnp.float32),
            pltpu.VMEM((128, HS), jnp.float32),
            pltpu.VMEM((40, 16), jnp.float32),
            pltpu.VMEM((16,), jnp.float32),
            pltpu.VMEM((16, 16), jnp.float32),
            pltpu.VMEM((5, 128), jnp.int32),
            pltpu.SemaphoreType.DMA,
            pltpu.SemaphoreType.DMA,
            pltpu.VMEM_SHARED((NP // 16, 16), jnp.float32),
            pltpu.VMEM_SHARED((16, 16), jnp.float32),
            pltpu.VMEM_SHARED((NP, HS), jnp.float32),
        ],
    )


_edge_kernel_l1 = _make_edge_kernel(32, True)
_edge_kernel_l2 = _make_edge_kernel(16, False)

_tc1 = pl.pallas_call(
    _tc1_body,
    out_shape=[
        jax.ShapeDtypeStruct((NP, 64), jnp.float32),
        jax.ShapeDtypeStruct((NP,), jnp.float32),
        jax.ShapeDtypeStruct((NP,), jnp.float32),
        jax.ShapeDtypeStruct((E,), jnp.float32),
    ],
)

_tc2 = pl.pallas_call(
    _tc2_body,
    out_shape=[
        jax.ShapeDtypeStruct((NP, 16), jnp.float32),
        jax.ShapeDtypeStruct((NP,), jnp.float32),
        jax.ShapeDtypeStruct((NP,), jnp.float32),
    ],
)

_tc3 = pl.pallas_call(
    _tc3_body,
    out_shape=jax.ShapeDtypeStruct((NP, 16), jnp.float32),
)


def kernel(data, edge_index, edge_weight, W1, att_src1, att_dst1, b1,
           W2, att_src2, att_dst2, b2):
    xp = jnp.pad(data, ((0, NP - N), (0, 0)))

    h1, as1, ad1, la = _tc1(xp, W1, att_src1, att_dst1, edge_weight)

    loop = jnp.arange(N, dtype=jnp.int32)
    padi = jnp.zeros((EP - E - N,), dtype=jnp.int32)
    src = jnp.concatenate([edge_index[0], loop, padi])
    dst = jnp.concatenate([edge_index[1], loop, padi])
    la_full = jnp.concatenate([
        la, jnp.zeros((N,), jnp.float32),
        jnp.full((EP - E - N,), PAD_LA, jnp.float32),
    ])

    rid = jnp.arange(NP // 16, dtype=jnp.int32).reshape(5, 128)
    src2 = src.reshape(EP // 128, 128)
    dst2 = dst.reshape(EP // 128, 128)
    h1n = h1[:N]
    p1 = _edge_kernel_l1(src2, dst2, la_full, as1, ad1,
                         h1n[:, :32], h1n[:, 32:], rid)
    agg1 = jnp.concatenate([p1[0], p1[1]], axis=1)
    h2, as2, ad2 = _tc2(agg1, b1, W2, att_src2, att_dst2)
    h2n = h2[:N]
    p2 = _edge_kernel_l2(src2, dst2, la_full, as2, ad2, h2n, h2n, rid)
    out = _tc3(p2, b2)
    return out[:N]


# speedup vs baseline: 40.2585x; 1.0500x over previous
import functools

import jax
import jax.numpy as jnp
from jax import lax
from jax.experimental import pallas as pl
from jax.experimental.pallas import tpu as pltpu
from jax.experimental.pallas import tpu_sc as plsc

N = 10000
NP = 10240
E = 320000
EP = 344064
S12 = EP // 16
S3 = EP // 32
CH = 128
NEG = 0.2
PAD_LA = -1e30


def _tc1_body(x_ref, w_ref, avs_ref, avd_ref, ew_ref,
              h_ref, as_ref, ad_ref, la_ref):
    h = jnp.dot(x_ref[...], w_ref[...], preferred_element_type=jnp.float32)
    h_ref[...] = h
    as_ref[...] = jnp.sum(h * avs_ref[...][None, :], axis=1)
    ad_ref[...] = jnp.sum(h * avd_ref[...][None, :], axis=1)
    la_ref[...] = jnp.log2(ew_ref[...])


def _tc2_body(p_ref, dn_ref, b_ref, w_ref, avs_ref, avd_ref,
              h_ref, as_ref, ad_ref):
    o = p_ref[...] / (dn_ref[...] + 1e-16) + b_ref[...][None, :]
    o = jnp.maximum(o, 0.0)
    h = jnp.dot(o, w_ref[...], preferred_element_type=jnp.float32)
    h_ref[...] = h
    as_ref[...] = jnp.sum(h * avs_ref[...][None, :], axis=1)
    ad_ref[...] = jnp.sum(h * avd_ref[...][None, :], axis=1)


def _tc3_body(p_ref, dn_ref, b_ref, o_ref):
    o_ref[...] = ((p_ref[0] + p_ref[1]) / (dn_ref[...] + 1e-16)
                  + b_ref[...][None, :])


def _edge_body(HS, split_cols, src_hbm, dst_hbm, la_hbm, as_hbm, ad_hbm,
               ha_hbm, hb_hbm, rid_hbm, out_hbm, dn_hbm,
               as_tab, ad_tab, dn_tab, src_buf, dst_buf, la_buf,
               rows0, rows1, rows2, zrows, zbuf, rid_v,
               gsem0, gsem1, gsem2, ssem0, ssem1, ssem2, sem3,
               dn_sh, out_sh):
    c = lax.axis_index("c")
    t = lax.axis_index("s")
    JH = HS // 16
    zero16 = jnp.zeros((16,), jnp.float32)

    with jax.named_scope("p0_stage"):
        nrow = S12 // 128
        pltpu.async_copy(as_hbm, as_tab, sem3)
        pltpu.async_copy(ad_hbm, ad_tab, sem3)
        pltpu.async_copy(rid_hbm, rid_v, sem3)
        pltpu.async_copy(src_hbm.at[pl.ds(t * nrow, nrow)], src_buf, sem3)
        pltpu.async_copy(dst_hbm.at[pl.ds(t * nrow, nrow)], dst_buf, sem3)
        pltpu.async_copy(la_hbm.at[pl.ds(t * S12, S12)], la_buf, sem3)

        def _zb(i, carry):
            zbuf[i, pl.ds(0, 16)] = zero16
            return carry
        lax.fori_loop(0, 40, _zb, 0)

        def _zr(i, carry):
            e = i // JH
            j = i % JH
            zrows[e, pl.ds(j * 16, 16)] = zero16
            return carry
        lax.fori_loop(0, 32 * JH, _zr, 0)

        def _zd(i, carry):
            dn_tab[i, pl.ds(0, 16)] = zero16
            return carry
        lax.fori_loop(0, NP // 16, _zd, 0)

        pltpu.sync_copy(zbuf, dn_sh.at[pl.ds(t * 40, 40)])
        for k in range(20):
            pltpu.sync_copy(zrows.at[pl.ds(0, 32)],
                            out_sh.at[pl.ds(t * 640 + k * 32, 32)])

        pltpu.make_async_copy(as_hbm, as_tab, sem3).wait()
        pltpu.make_async_copy(ad_hbm, ad_tab, sem3).wait()
        pltpu.make_async_copy(rid_hbm, rid_v, sem3).wait()
        pltpu.make_async_copy(src_hbm.at[pl.ds(t * nrow, nrow)],
                              src_buf, sem3).wait()
        pltpu.make_async_copy(dst_hbm.at[pl.ds(t * nrow, nrow)],
                              dst_buf, sem3).wait()
        pltpu.make_async_copy(la_hbm.at[pl.ds(t * S12, S12)],
                              la_buf, sem3).wait()

    with jax.named_scope("p1_bound"):
        def _mx(i, m):
            a = jnp.maximum(as_tab[pl.ds(i * 128, 16)],
                            ad_tab[pl.ds(i * 128, 16)])
            for c8 in range(1, 8):
                a = jnp.maximum(a, as_tab[pl.ds(i * 128 + c8 * 16, 16)])
                a = jnp.maximum(a, ad_tab[pl.ds(i * 128 + c8 * 16, 16)])
            return jnp.maximum(m, a)
        mboth = lax.fori_loop(0, NP // 128, _mx,
                              jnp.full((16,), -3e38, jnp.float32))
        sb = 2.0 * jnp.max(mboth)
        shift = jnp.where(sb >= 0.0, sb, NEG * sb)
        gv = jnp.full((16,), shift, jnp.float32)

    with jax.named_scope("p2_exp"):
        def _p2(r, carry):
            for c8 in range(8):
                sv = src_buf[r, pl.ds(c8 * 16, 16)]
                dv = dst_buf[r, pl.ds(c8 * 16, 16)]
                x = (plsc.load_gather(as_tab, [sv])
                     + plsc.load_gather(ad_tab, [dv]))
                x = jnp.where(x >= 0.0, x, NEG * x)
                ex = jnp.exp(x + la_buf[pl.ds(r * 128 + c8 * 16, 16)] - gv)
                la_buf[pl.ds(r * 128 + c8 * 16, 16)] = ex
                plsc.addupdate_scatter(dn_tab, [dv >> 4, dv & 15], ex)
            return carry
        lax.fori_loop(0, nrow, _p2, 0)

    with jax.named_scope("b1_barrier"):
        plsc.subcore_barrier()

    with jax.named_scope("p2_reduce"):
        for k in range(5):
            pltpu.sync_copy(dn_tab.at[pl.ds(k * 128, 128)],
                            dn_sh.at[rid_v.at[k]], add=True)

    with jax.named_scope("p3_rows"):
        off0 = 0 if split_cols else c * (S3 // CH)
        nch = (S12 // CH) if split_cols else (S3 // CH)
        h_table = [ha_hbm, hb_hbm]
        rbufs = [rows0, rows1, rows2]
        gsems = [gsem0, gsem1, gsem2]
        ssems = [ssem0, ssem1, ssem2]

        def _gather_desc(r3, b):
            return pltpu.make_async_copy(h_table[0].at[src_buf.at[r3]],
                                         rbufs[b], gsems[b])

        def _issue(r3, b):
            @pl.when(c == 0)
            def _g0():
                pltpu.async_copy(h_table[0].at[src_buf.at[r3]],
                                 rbufs[b], gsems[b])

            @pl.when(c == 1)
            def _g1():
                pltpu.async_copy(h_table[1].at[src_buf.at[r3]],
                                 rbufs[b], gsems[b])

        def _scat_desc(r3, b):
            return pltpu.make_async_copy(rbufs[b], out_sh.at[dst_buf.at[r3]],
                                         ssems[b])

        _issue(off0, 0)
        _issue(off0 + 1, 1)

        def _p3(k4, carry):
            for l in range(3):
                k = k4 * 3 + l
                b2 = (l + 2) % 3

                @pl.when(k + 2 < nch)
                def _ig():
                    @pl.when(k >= 1)
                    def _ws():
                        _scat_desc(off0 + k - 1, b2).wait()
                    _issue(off0 + k + 2, b2)

                r3 = off0 + k
                _gather_desc(r3, l).wait()

                def _sc(v, carry2):
                    cv16 = la_buf[pl.ds(r3 * 128 + v * 16, 16)]
                    for q in range(16):
                        e = v * 16 + q
                        cv = jnp.full((16,), cv16[q], jnp.float32)
                        for j in range(JH):
                            rbufs[l][e, pl.ds(j * 16, 16)] = (
                                rbufs[l][e, pl.ds(j * 16, 16)] * cv)
                    return carry2
                lax.fori_loop(0, CH // 16, _sc, 0)

                _scat_desc(r3, l).start(add=True)
            return carry
        lax.fori_loop(0, nch // 3, _p3, 0)

        for i in range(3):
            _scat_desc(off0 + nch - 3 + i, (nch - 3 + i) % 3).wait()

    with jax.named_scope("b3_barrier"):
        plsc.subcore_barrier()

    with jax.named_scope("p4_out"):
        pltpu.sync_copy(out_sh.at[pl.ds(t * 640, 640)],
                        out_hbm.at[c, pl.ds(t * 640, 640)])

        @pl.when(c == 0)
        def _dn_out():
            pltpu.sync_copy(dn_sh.at[pl.ds(t * 40, 40)],
                            dn_hbm.at[pl.ds(t * 40, 40)])


def _make_edge_kernel(HS, split_cols):
    mesh = plsc.VectorSubcoreMesh(core_axis_name="c", subcore_axis_name="s")
    return pl.kernel(
        functools.partial(_edge_body, HS, split_cols),
        out_type=[jax.ShapeDtypeStruct((2, NP, HS), jnp.float32),
                  jax.ShapeDtypeStruct((NP // 16, 16), jnp.float32)],
        mesh=mesh,
        compiler_params=pltpu.CompilerParams(needs_layout_passes=False, use_tc_tiling_on_sc=False),
        scratch_types=[
            pltpu.VMEM((NP,), jnp.float32),
            pltpu.VMEM((NP,), jnp.float32),
            pltpu.VMEM((NP // 16, 16), jnp.float32),
            pltpu.VMEM((S12 // 128, 128), jnp.int32),
            pltpu.VMEM((S12 // 128, 128), jnp.int32),
            pltpu.VMEM((S12,), jnp.float32),
            pltpu.VMEM((CH, HS), jnp.float32),
            pltpu.VMEM((CH, HS), jnp.float32),
            pltpu.VMEM((CH, HS), jnp.float32),
            pltpu.VMEM((32, HS), jnp.float32),
            pltpu.VMEM((40, 16), jnp.float32),
            pltpu.VMEM((5, 128), jnp.int32),
            pltpu.SemaphoreType.DMA,
            pltpu.SemaphoreType.DMA,
            pltpu.SemaphoreType.DMA,
            pltpu.SemaphoreType.DMA,
            pltpu.SemaphoreType.DMA,
            pltpu.SemaphoreType.DMA,
            pltpu.SemaphoreType.DMA,
            pltpu.VMEM_SHARED((NP // 16, 16), jnp.float32),
            pltpu.VMEM_SHARED((NP, HS), jnp.float32),
        ],
    )


_edge_kernel_l1 = _make_edge_kernel(32, True)
_edge_kernel_l2 = _make_edge_kernel(16, False)

_tc1 = pl.pallas_call(
    _tc1_body,
    out_shape=[
        jax.ShapeDtypeStruct((NP, 64), jnp.float32),
        jax.ShapeDtypeStruct((NP,), jnp.float32),
        jax.ShapeDtypeStruct((NP,), jnp.float32),
        jax.ShapeDtypeStruct((E,), jnp.float32),
    ],
)

_tc2 = pl.pallas_call(
    _tc2_body,
    out_shape=[
        jax.ShapeDtypeStruct((NP, 16), jnp.float32),
        jax.ShapeDtypeStruct((NP,), jnp.float32),
        jax.ShapeDtypeStruct((NP,), jnp.float32),
    ],
)

_tc3 = pl.pallas_call(
    _tc3_body,
    out_shape=jax.ShapeDtypeStruct((NP, 16), jnp.float32),
)


def kernel(data, edge_index, edge_weight, W1, att_src1, att_dst1, b1,
           W2, att_src2, att_dst2, b2):
    xp = jnp.pad(data, ((0, NP - N), (0, 0)))

    h1, as1, ad1, la = _tc1(xp, W1, att_src1, att_dst1, edge_weight)

    loop = jnp.arange(N, dtype=jnp.int32)
    padi = jnp.zeros((EP - E - N,), dtype=jnp.int32)
    src = jnp.concatenate([edge_index[0], loop, padi])
    dst = jnp.concatenate([edge_index[1], loop, padi])
    la_full = jnp.concatenate([
        la, jnp.zeros((N,), jnp.float32),
        jnp.full((EP - E - N,), PAD_LA, jnp.float32),
    ])

    rid = jnp.arange(NP // 16, dtype=jnp.int32).reshape(5, 128)
    src2 = src.reshape(EP // 128, 128)
    dst2 = dst.reshape(EP // 128, 128)
    h1n = h1[:N]
    p1, dn1 = _edge_kernel_l1(src2, dst2, la_full, as1, ad1,
                              h1n[:, :32], h1n[:, 32:], rid)
    agg1 = jnp.concatenate([p1[0], p1[1]], axis=1)
    h2, as2, ad2 = _tc2(agg1, dn1.reshape(NP, 1), b1, W2, att_src2, att_dst2)
    h2n = h2[:N]
    p2, dn2 = _edge_kernel_l2(src2, dst2, la_full, as2, ad2, h2n, h2n, rid)
    out = _tc3(p2, dn2.reshape(NP, 1), b2)
    return out[:N]
